# Initial kernel scaffold; baseline (speedup 1.0000x reference)
import jax
import jax.numpy as jnp
from jax.experimental import pallas as pl


def kernel(days_seqs, emb_weight):
    raise NotImplementedError("write your pallas kernel here")


# speedup vs baseline: 3.4676x; 3.4676x over previous
import functools

import jax
import jax.numpy as jnp
from jax import lax
from jax.experimental import pallas as pl
from jax.experimental.pallas import tpu as pltpu
from jax.experimental.pallas import tpu_sc as plsc

NUM_DAYS = 732
HIDDEN = 64
BATCH = 4096
HIST_LEN = 200

B = BATCH * HIST_LEN
NC, NS = 2, 16
NW = NC * NS
CHUNK = 128
BPW = B // NW
NCH = BPW // CHUNK
NPAIR = NCH // 2

_mesh = plsc.VectorSubcoreMesh(core_axis_name="c", subcore_axis_name="s")


@functools.partial(
    pl.kernel,
    mesh=_mesh,
    compiler_params=pltpu.CompilerParams(use_tc_tiling_on_sc=False),
    out_type=jax.ShapeDtypeStruct((B, HIDDEN), jnp.float32),
    scratch_types=[
        pltpu.VMEM((NCH, CHUNK), jnp.int32),
        pltpu.VMEM((2, CHUNK, HIDDEN), jnp.float32),
        pltpu.SemaphoreType.DMA,
        pltpu.SemaphoreType.DMA,
        pltpu.SemaphoreType.DMA,
        pltpu.SemaphoreType.DMA,
    ],
)
def _emb_lookup(idx_hbm, table_hbm, out_hbm, idx_v, rows_v, sg0, sg1, ss0, ss1):
    wid = lax.axis_index("s") * NC + lax.axis_index("c")
    row_base = wid * NCH
    out_base = wid * BPW

    pltpu.sync_copy(idx_hbm.at[pl.ds(row_base, NCH)], idx_v)

    sg = (sg0, sg1)
    ss = (ss0, ss1)

    def gather(i, slot):
        return pltpu.make_async_copy(
            table_hbm.at[idx_v.at[i]], rows_v.at[slot], sg[slot])

    def store(i, slot):
        return pltpu.make_async_copy(
            rows_v.at[slot],
            out_hbm.at[pl.ds(out_base + i * CHUNK, CHUNK)],
            ss[slot])

    gather(0, 0).start()

    def pair(p, _):
        i0 = p * 2
        i1 = i0 + 1
        @pl.when(p > 0)
        def _():
            store(i0 - 1, 1).wait()
        gather(i1, 1).start()
        gather(i0, 0).wait()
        store(i0, 0).start()
        @pl.when(p < NPAIR - 1)
        def _():
            store(i0, 0).wait()
            gather(i0 + 2, 0).start()
        gather(i1, 1).wait()
        store(i1, 1).start()
        return 0

    lax.fori_loop(0, NPAIR, pair, 0)
    store(NCH - 2, 0).wait()
    store(NCH - 1, 1).wait()


def kernel(days_seqs, emb_weight):
    idx = days_seqs.reshape(B // CHUNK, CHUNK)
    out = _emb_lookup(idx, emb_weight)
    return out.reshape(BATCH, HIST_LEN, HIDDEN)


# speedup vs baseline: 3.4739x; 1.0018x over previous
import functools

import jax
import jax.numpy as jnp
from jax import lax
from jax.experimental import pallas as pl
from jax.experimental.pallas import tpu as pltpu
from jax.experimental.pallas import tpu_sc as plsc

NUM_DAYS = 732
HIDDEN = 64
BATCH = 4096
HIST_LEN = 200

NC, NS = 2, 16
NW = NC * NS
RPW = BATCH // NW
CA = 128
CB = HIST_LEN - CA

_mesh = plsc.VectorSubcoreMesh(core_axis_name="c", subcore_axis_name="s")


@functools.partial(
    pl.kernel,
    mesh=_mesh,
    compiler_params=pltpu.CompilerParams(use_tc_tiling_on_sc=False),
    out_type=jax.ShapeDtypeStruct((BATCH, HIST_LEN, HIDDEN), jnp.float32),
    scratch_types=[
        pltpu.VMEM((RPW, HIST_LEN), jnp.int32),
        pltpu.VMEM((2, CA, HIDDEN), jnp.float32),
        pltpu.VMEM((2, CB, HIDDEN), jnp.float32),
        pltpu.SemaphoreType.DMA,
        pltpu.SemaphoreType.DMA,
        pltpu.SemaphoreType.DMA,
        pltpu.SemaphoreType.DMA,
    ],
)
def _emb_lookup(idx_hbm, table_hbm, out_hbm, idx_v, rows_a, rows_b,
                sg0, sg1, ss0, ss1):
    wid = lax.axis_index("s") * NC + lax.axis_index("c")
    row_base = wid * RPW

    pltpu.sync_copy(idx_hbm.at[pl.ds(row_base, RPW)], idx_v)

    sg = (sg0, sg1)
    ss = (ss0, ss1)

    def gather_a(r, slot):
        return pltpu.make_async_copy(
            table_hbm.at[idx_v.at[r, pl.ds(0, CA)]], rows_a.at[slot], sg[slot])

    def gather_b(r, slot):
        return pltpu.make_async_copy(
            table_hbm.at[idx_v.at[r, pl.ds(CA, CB)]], rows_b.at[slot], sg[slot])

    def store_a(r, slot):
        return pltpu.make_async_copy(
            rows_a.at[slot], out_hbm.at[row_base + r, pl.ds(0, CA)], ss[slot])

    def store_b(r, slot):
        return pltpu.make_async_copy(
            rows_b.at[slot], out_hbm.at[row_base + r, pl.ds(CA, CB)], ss[slot])

    gather_a(0, 0).start()
    gather_b(0, 0).start()

    def pair(p, _):
        r0 = p * 2
        r1 = r0 + 1
        @pl.when(p > 0)
        def _():
            store_a(r0 - 1, 1).wait()
            store_b(r0 - 1, 1).wait()
        gather_a(r1, 1).start()
        gather_b(r1, 1).start()
        gather_a(r0, 0).wait()
        store_a(r0, 0).start()
        gather_b(r0, 0).wait()
        store_b(r0, 0).start()
        @pl.when(p < RPW // 2 - 1)
        def _():
            store_a(r0, 0).wait()
            store_b(r0, 0).wait()
            gather_a(r0 + 2, 0).start()
            gather_b(r0 + 2, 0).start()
        gather_a(r1, 1).wait()
        store_a(r1, 1).start()
        gather_b(r1, 1).wait()
        store_b(r1, 1).start()
        return 0

    lax.fori_loop(0, RPW // 2, pair, 0)
    store_a(RPW - 2, 0).wait()
    store_b(RPW - 2, 0).wait()
    store_a(RPW - 1, 1).wait()
    store_b(RPW - 1, 1).wait()


def kernel(days_seqs, emb_weight):
    return _emb_lookup(days_seqs, emb_weight)


# speedup vs baseline: 4.8817x; 1.4052x over previous
import functools

import jax
import jax.numpy as jnp
from jax import lax
from jax.experimental import pallas as pl
from jax.experimental.pallas import tpu as pltpu
from jax.experimental.pallas import tpu_sc as plsc

NUM_DAYS = 732
HIDDEN = 64
BATCH = 4096
HIST_LEN = 200

NC, NS = 2, 16
NW = NC * NS
NHG = HIDDEN // 8
NTQ = NW // NHG
TPW = HIST_LEN // NTQ
NPAIR = TPW // 2
NBCH = BATCH // 16
NROW = HIST_LEN * HIDDEN
NELT = NROW * BATCH
BLK = 8 * BATCH
TRPW = NROW // 8 // NW

_mesh = plsc.VectorSubcoreMesh(core_axis_name="c", subcore_axis_name="s")


@functools.partial(
    pl.kernel,
    mesh=_mesh,
    compiler_params=pltpu.CompilerParams(
        use_tc_tiling_on_sc=False, needs_layout_passes=False),
    out_type=jax.ShapeDtypeStruct((NELT,), jnp.float32),
    scratch_types=[
        pltpu.VMEM((HIDDEN, NUM_DAYS), jnp.float32),
        pltpu.VMEM((2, BATCH), jnp.int32),
        pltpu.VMEM((2, BLK), jnp.float32),
        pltpu.SemaphoreType.DMA,
        pltpu.SemaphoreType.DMA,
        pltpu.SemaphoreType.DMA,
        pltpu.SemaphoreType.DMA,
    ],
)
def _emb_gather(idx_hbm, table_hbm, out_hbm, table_v, idx_v, rows_v,
                si0, si1, so0, so1):
    wid = lax.axis_index("s") * NC + lax.axis_index("c")
    h0 = lax.rem(wid, NHG) * 8
    t0 = lax.div(wid, NHG) * TPW

    pltpu.sync_copy(table_hbm, table_v)

    si = (si0, si1)
    so = (so0, so1)

    def idx_load(j, slot):
        return pltpu.make_async_copy(
            idx_hbm.at[pl.ds((t0 + j) * BATCH, BATCH)], idx_v.at[slot],
            si[slot])

    def store(j, slot):
        return pltpu.make_async_copy(
            rows_v.at[slot],
            out_hbm.at[pl.ds(((t0 + j) * HIDDEN + h0) * BATCH, BLK)],
            so[slot])

    def compute(slot):
        def chunk(c, _):
            days = idx_v.at[slot][pl.ds(c * 16, 16)]
            for hh in range(8):
                hrow = h0 + jnp.full((16,), hh, jnp.int32)
                v = plsc.load_gather(table_v, [hrow, days])
                rows_v.at[slot][pl.ds(hh * BATCH + c * 16, 16)] = v
            return 0
        lax.fori_loop(0, NBCH, chunk, 0)

    idx_load(0, 0).start()
    idx_load(1, 1).start()
    idx_load(0, 0).wait()
    compute(0)
    store(0, 0).start()
    idx_load(2, 0).start()
    idx_load(1, 1).wait()
    compute(1)
    store(1, 1).start()

    def pair(p, _):
        j0 = p * 2
        j1 = j0 + 1
        idx_load(j1, 1).start()
        store(j0 - 2, 0).wait()
        idx_load(j0, 0).wait()
        compute(0)
        store(j0, 0).start()
        idx_load(j0 + 2, 0).start()
        store(j1 - 2, 1).wait()
        idx_load(j1, 1).wait()
        compute(1)
        store(j1, 1).start()
        return 0

    lax.fori_loop(1, NPAIR - 1, pair, 0)

    j0 = TPW - 2
    j1 = TPW - 1
    idx_load(j1, 1).start()
    store(j0 - 2, 0).wait()
    idx_load(j0, 0).wait()
    compute(0)
    store(j0, 0).start()
    store(j1 - 2, 1).wait()
    idx_load(j1, 1).wait()
    compute(1)
    store(j1, 1).start()
    store(j0, 0).wait()
    store(j1, 1).wait()


@functools.partial(
    pl.kernel,
    mesh=_mesh,
    out_type=jax.ShapeDtypeStruct((NROW, BATCH), jnp.float32),
    scratch_types=[
        pltpu.VMEM((2, 8, BATCH), jnp.float32),
        pltpu.SemaphoreType.DMA,
        pltpu.SemaphoreType.DMA,
        pltpu.SemaphoreType.DMA,
        pltpu.SemaphoreType.DMA,
    ],
)
def _retile(flat_hbm, out_hbm, buf_v, li0, li1, lo0, lo1):
    wid = lax.axis_index("s") * NC + lax.axis_index("c")
    r0 = wid * TRPW

    li = (li0, li1)
    lo = (lo0, lo1)

    def load(k, slot):
        return [pltpu.make_async_copy(
            flat_hbm.at[pl.ds(((r0 + k) * 8 + i) * BATCH, BATCH)],
            buf_v.at[slot, i], li[slot]) for i in range(8)]

    def store(k, slot):
        return pltpu.make_async_copy(
            buf_v.at[slot], out_hbm.at[pl.ds((r0 + k) * 8, 8)], lo[slot])

    def start_load(k, slot):
        for c in load(k, slot):
            c.start()

    def wait_load(k, slot):
        for c in load(k, slot):
            c.wait()

    start_load(0, 0)
    start_load(1, 1)
    wait_load(0, 0)
    store(0, 0).start()
    wait_load(1, 1)
    store(1, 1).start()

    def pair(p, _):
        k0 = p * 2
        k1 = k0 + 1
        store(k0 - 2, 0).wait()
        start_load(k0, 0)
        store(k1 - 2, 1).wait()
        start_load(k1, 1)
        wait_load(k0, 0)
        store(k0, 0).start()
        wait_load(k1, 1)
        store(k1, 1).start()
        return 0

    lax.fori_loop(1, TRPW // 2, pair, 0)

    store(TRPW - 2, 0).wait()
    store(TRPW - 1, 1).wait()


def kernel(days_seqs, emb_weight):
    idx_flat = days_seqs.T.reshape(-1)
    flat = _emb_gather(idx_flat, emb_weight.T)
    out2d = _retile(flat)
    return out2d.reshape(HIST_LEN, HIDDEN, BATCH).transpose(2, 0, 1)


# speedup vs baseline: 5.3935x; 1.1048x over previous
import functools

import jax
import jax.numpy as jnp
from jax import lax
from jax.experimental import pallas as pl
from jax.experimental.pallas import tpu as pltpu
from jax.experimental.pallas import tpu_sc as plsc

NUM_DAYS = 732
HIDDEN = 64
BATCH = 4096
HIST_LEN = 200

NC, NS = 2, 16
NW = NC * NS
NHG = HIDDEN // 8
NTQ = NW // NHG
TPW = HIST_LEN // NTQ
NPAIR = TPW // 2
NBCH = BATCH // 16
NROW = HIST_LEN * HIDDEN

_mesh = plsc.VectorSubcoreMesh(core_axis_name="c", subcore_axis_name="s")


@functools.partial(
    pl.kernel,
    mesh=_mesh,
    compiler_params=pltpu.CompilerParams(needs_layout_passes=False),
    out_type=jax.ShapeDtypeStruct((NROW, BATCH), jnp.float32),
    scratch_types=[
        pltpu.VMEM((HIDDEN, NUM_DAYS), jnp.float32),
        pltpu.VMEM((BATCH,), jnp.int32),
        pltpu.VMEM((BATCH,), jnp.int32),
        pltpu.VMEM((8, BATCH), jnp.float32),
        pltpu.VMEM((8, BATCH), jnp.float32),
        pltpu.SemaphoreType.DMA,
        pltpu.SemaphoreType.DMA,
        pltpu.SemaphoreType.DMA,
        pltpu.SemaphoreType.DMA,
    ],
)
def _emb_lookup(idx_hbm, table_hbm, out_hbm, table_v, idx0, idx1,
                rows0, rows1, si0, si1, so0, so1):
    wid = lax.axis_index("s") * NC + lax.axis_index("c")
    h0 = lax.rem(wid, NHG) * 8
    t0 = lax.div(wid, NHG) * TPW

    pltpu.sync_copy(table_hbm, table_v)

    iv = (idx0, idx1)
    rv = (rows0, rows1)
    si = (si0, si1)
    so = (so0, so1)

    def idx_load(j, slot):
        return pltpu.make_async_copy(
            idx_hbm.at[pl.ds((t0 + j) * BATCH, BATCH)], iv[slot], si[slot])

    def store(j, slot):
        return pltpu.make_async_copy(
            rv[slot], out_hbm.at[pl.ds((t0 + j) * HIDDEN + h0, 8)], so[slot])

    hrows = [h0 + jnp.full((16,), hh, jnp.int32) for hh in range(8)]

    def compute(slot):
        def chunk(c, _):
            days = iv[slot][pl.ds(c * 16, 16)]
            for hh in range(8):
                v = plsc.load_gather(table_v, [hrows[hh], days])
                rv[slot][hh, pl.ds(c * 16, 16)] = v
            return 0
        lax.fori_loop(0, NBCH, chunk, 0)

    idx_load(0, 0).start()
    idx_load(1, 1).start()
    idx_load(0, 0).wait()
    compute(0)
    store(0, 0).start()
    idx_load(2, 0).start()
    idx_load(1, 1).wait()
    compute(1)
    store(1, 1).start()

    def pair(p, _):
        j0 = p * 2
        j1 = j0 + 1
        idx_load(j1, 1).start()
        store(j0 - 2, 0).wait()
        idx_load(j0, 0).wait()
        compute(0)
        store(j0, 0).start()
        idx_load(j0 + 2, 0).start()
        store(j1 - 2, 1).wait()
        idx_load(j1, 1).wait()
        compute(1)
        store(j1, 1).start()
        return 0

    lax.fori_loop(1, NPAIR - 1, pair, 0)

    j0 = TPW - 2
    j1 = TPW - 1
    idx_load(j1, 1).start()
    store(j0 - 2, 0).wait()
    idx_load(j0, 0).wait()
    compute(0)
    store(j0, 0).start()
    store(j1 - 2, 1).wait()
    idx_load(j1, 1).wait()
    compute(1)
    store(j1, 1).start()
    store(j0, 0).wait()
    store(j1, 1).wait()


def kernel(days_seqs, emb_weight):
    idx_flat = days_seqs.T.reshape(-1)
    out2d = _emb_lookup(idx_flat, emb_weight.T)
    return out2d.reshape(HIST_LEN, HIDDEN, BATCH).transpose(2, 0, 1)


# speedup vs baseline: 25.6555x; 4.7568x over previous
import functools

import jax
import jax.numpy as jnp
from jax import lax
from jax.experimental import pallas as pl
from jax.experimental.pallas import tpu as pltpu
from jax.experimental.pallas import tpu_sc as plsc

NUM_DAYS = 732
HIDDEN = 64
BATCH = 4096
HIST_LEN = 200

NC, NS = 2, 16
NW = NC * NS
NHG = HIDDEN // 8
NTQ = NW // NHG
TPW = HIST_LEN // NTQ
NPAIR = TPW // 2
NBCH = BATCH // 16
NROW = HIST_LEN * HIDDEN

_mesh = plsc.VectorSubcoreMesh(core_axis_name="c", subcore_axis_name="s")


@functools.partial(
    pl.kernel,
    mesh=_mesh,
    compiler_params=pltpu.CompilerParams(needs_layout_passes=False),
    out_type=jax.ShapeDtypeStruct((NROW, BATCH), jnp.float32),
    scratch_types=[
        pltpu.VMEM((HIDDEN, NUM_DAYS), jnp.float32),
        pltpu.VMEM((BATCH,), jnp.int32),
        pltpu.VMEM((BATCH,), jnp.int32),
        pltpu.VMEM((8, BATCH), jnp.float32),
        pltpu.VMEM((8, BATCH), jnp.float32),
        pltpu.SemaphoreType.DMA,
        pltpu.SemaphoreType.DMA,
        pltpu.SemaphoreType.DMA,
        pltpu.SemaphoreType.DMA,
    ],
)
def _emb_lookup(idx_hbm, table_hbm, out_hbm, table_v, idx0, idx1,
                rows0, rows1, si0, si1, so0, so1):
    wid = lax.axis_index("s") * NC + lax.axis_index("c")
    h0 = lax.rem(wid, NHG) * 8
    t0 = lax.div(wid, NHG) * TPW

    pltpu.sync_copy(table_hbm, table_v)

    iv = (idx0, idx1)
    rv = (rows0, rows1)
    si = (si0, si1)
    so = (so0, so1)

    def idx_load(j, slot):
        return pltpu.make_async_copy(
            idx_hbm.at[pl.ds((t0 + j) * BATCH, BATCH)], iv[slot], si[slot])

    def store(j, slot):
        return pltpu.make_async_copy(
            rv[slot], out_hbm.at[pl.ds((t0 + j) * HIDDEN + h0, 8)], so[slot])

    hrows = [h0 + jnp.full((16,), hh, jnp.int32) for hh in range(8)]

    def compute(slot):
        @functools.partial(plsc.parallel_loop, 0, NBCH, unroll=8)
        def _(c):
            days = iv[slot][pl.ds(c * 16, 16)]
            for hh in range(8):
                v = plsc.load_gather(table_v, [hrows[hh], days])
                rv[slot][hh, pl.ds(c * 16, 16)] = v

    idx_load(0, 0).start()
    idx_load(1, 1).start()
    idx_load(0, 0).wait()
    compute(0)
    store(0, 0).start()
    idx_load(2, 0).start()
    idx_load(1, 1).wait()
    compute(1)
    store(1, 1).start()

    def pair(p, _):
        j0 = p * 2
        j1 = j0 + 1
        idx_load(j1, 1).start()
        store(j0 - 2, 0).wait()
        idx_load(j0, 0).wait()
        compute(0)
        store(j0, 0).start()
        idx_load(j0 + 2, 0).start()
        store(j1 - 2, 1).wait()
        idx_load(j1, 1).wait()
        compute(1)
        store(j1, 1).start()
        return 0

    lax.fori_loop(1, NPAIR - 1, pair, 0)

    j0 = TPW - 2
    j1 = TPW - 1
    idx_load(j1, 1).start()
    store(j0 - 2, 0).wait()
    idx_load(j0, 0).wait()
    compute(0)
    store(j0, 0).start()
    store(j1 - 2, 1).wait()
    idx_load(j1, 1).wait()
    compute(1)
    store(j1, 1).start()
    store(j0, 0).wait()
    store(j1, 1).wait()


def kernel(days_seqs, emb_weight):
    idx_flat = days_seqs.T.reshape(-1)
    out2d = _emb_lookup(idx_flat, emb_weight.T)
    return out2d.reshape(HIST_LEN, HIDDEN, BATCH).transpose(2, 0, 1)
